# baseline (speedup 1.0000x reference)
import functools

import numpy as np
import jax
from jax import lax
import jax.numpy as jnp
from jax.experimental import pallas as pl
from jax.experimental.pallas import tpu as pltpu
from jax.experimental.pallas import tpu_sc as plsc

_N = 512
_D = 4096
_CHUNK = 2048
_NCHUNKS = _D // _CHUNK
_SLOTS = 3
_NSUB = 16
_ROWS_PER = _N // _NSUB
_LANES = 16


def _triplet_columns(n):
    labels = list(range(int(n / 2))) + list(range(int(n / 2)))
    rng = np.random.RandomState(0)
    triplets = []
    for i in range(len(labels)):
        triplets_i = []
        for j in range(len(labels)):
            if labels[i] == labels[j] and i != j:
                for k in range(len(labels)):
                    if labels[i] != labels[k]:
                        triplets_i.append([i, j, k])
        rng.shuffle(triplets_i)
        triplets += triplets_i[:3]
    trip = np.asarray(triplets, dtype=np.int32)
    jj = np.zeros((_SLOTS, n), dtype=np.int32)
    kk = np.zeros((_SLOTS, n), dtype=np.int32)
    fill = np.zeros((n,), dtype=np.int64)
    for (i, j, k) in trip:
        m = fill[i]
        jj[m, i] = j
        kk[m, i] = k
        fill[i] += 1
    assert (fill == _SLOTS).all()
    return jj, kk, trip.shape[0]


_JJ, _KK, _NUM_TRIPLETS = _triplet_columns(_N)
_IDX_SC = np.zeros((_NSUB, 8, _ROWS_PER), dtype=np.int32)
for _m in range(_SLOTS):
    _IDX_SC[:, _m, :] = _JJ[_m].reshape(_NSUB, _ROWS_PER)
    _IDX_SC[:, 4 + _m, :] = _KK[_m].reshape(_NSUB, _ROWS_PER)
_IDX_SC_FLAT = _IDX_SC.reshape(-1)
_IDX_PER_SUB = 8 * _ROWS_PER


def _dist_kernel(x_ref, out_ref, acc_ref):
    d = pl.program_id(0)
    xc = x_ref[...].astype(jnp.bfloat16)
    part = jax.lax.dot_general(
        xc, xc,
        dimension_numbers=(((1,), (1,)), ((), ())),
        preferred_element_type=jnp.float32,
    )

    @pl.when(d == 0)
    def _init():
        acc_ref[...] = part

    @pl.when(d > 0)
    def _accum():
        acc_ref[...] += part

    @pl.when(d == _NCHUNKS - 1)
    def _epilogue():
        gram = acc_ref[...]
        rows = jax.lax.broadcasted_iota(jnp.int32, (_N, _N), 0)
        cols = jax.lax.broadcasted_iota(jnp.int32, (_N, _N), 1)
        diag = jnp.where(rows == cols, gram, 0.0)
        xn_col = jnp.sum(diag, axis=1, keepdims=True)
        xn_row = jnp.sum(diag, axis=0, keepdims=True)
        out_ref[...] = jnp.maximum(xn_col + xn_row - 2.0 * gram, 0.0)


_sc_mesh = plsc.VectorSubcoreMesh(
    core_axis_name="c", subcore_axis_name="s", num_cores=1, num_subcores=_NSUB
)


@functools.partial(
    pl.kernel,
    out_type=(
        jax.ShapeDtypeStruct((_NSUB, _LANES), jnp.float32),
        jax.ShapeDtypeStruct((_LANES,), jnp.float32),
    ),
    mesh=_sc_mesh,
    scratch_types=[
        pltpu.VMEM((_ROWS_PER * _N,), jnp.float32),
        pltpu.VMEM((_IDX_PER_SUB,), jnp.int32),
        pltpu.VMEM((_LANES,), jnp.float32),
        pltpu.VMEM((_NSUB, _LANES), jnp.float32),
        pltpu.VMEM((_LANES,), jnp.float32),
    ],
    compiler_params=pltpu.CompilerParams(needs_layout_passes=False),
)
def _sc_loss(dist_hbm, idx_hbm, stage_hbm, out_hbm, rows_v, idx_v, acc_v, all_v, out_v):
    s = lax.axis_index("s")
    pltpu.sync_copy(dist_hbm.at[pl.ds(s * _ROWS_PER * _N, _ROWS_PER * _N)], rows_v)
    pltpu.sync_copy(idx_hbm.at[pl.ds(s * _IDX_PER_SUB, _IDX_PER_SUB)], idx_v)

    acc = jnp.zeros((_LANES,), jnp.float32)
    for g in range(_ROWS_PER // _LANES):
        row_base = (lax.iota(jnp.int32, _LANES) + g * _LANES) * _N
        for m in range(_SLOTS):
            jj = idx_v[pl.ds(m * _ROWS_PER + g * _LANES, _LANES)]
            kk = idx_v[pl.ds((4 + m) * _ROWS_PER + g * _LANES, _LANES)]
            dj = plsc.load_gather(rows_v, [row_base + jj])
            dk = plsc.load_gather(rows_v, [row_base + kk])
            delta = dj - dk
            t = jnp.exp(-jnp.abs(delta))
            z = t / (2.0 + t)
            z2 = z * z
            log1p = 2.0 * z * (1.0 + z2 * (1.0 / 3.0 + z2 * 0.2))
            acc = acc + jnp.maximum(delta, 0.0) + log1p

    acc_v[...] = acc
    pltpu.sync_copy(acc_v, stage_hbm.at[s])
    plsc.subcore_barrier()

    @pl.when(s == 0)
    def _reduce():
        pltpu.sync_copy(stage_hbm, all_v)
        tot = jnp.zeros((_LANES,), jnp.float32)
        for r in range(_NSUB):
            tot = tot + all_v[r, :]
        loss = jnp.sum(tot, axis=0) * (1.0 / float(_NUM_TRIPLETS))
        out_v[...] = jnp.full((_LANES,), loss, dtype=jnp.float32)
        pltpu.sync_copy(out_v, out_hbm)


@jax.jit
def kernel(x):
    dist = pl.pallas_call(
        _dist_kernel,
        grid=(_NCHUNKS,),
        out_shape=jax.ShapeDtypeStruct((_N, _N), jnp.float32),
        in_specs=[pl.BlockSpec((_N, _CHUNK), lambda d: (0, d))],
        out_specs=pl.BlockSpec((_N, _N), lambda d: (0, 0)),
        scratch_shapes=[pltpu.VMEM((_N, _N), jnp.float32)],
    )(x)
    _, out16 = _sc_loss(dist.reshape(-1), jnp.asarray(_IDX_SC_FLAT))
    return out16[0:1]


# speedup vs baseline: 1.0962x; 1.0962x over previous
import functools

import numpy as np
import jax
from jax import lax
import jax.numpy as jnp
from jax.experimental import pallas as pl
from jax.experimental.pallas import tpu as pltpu
from jax.experimental.pallas import tpu_sc as plsc

_N = 512
_D = 4096
_CHUNK = 2048
_NCHUNKS = _D // _CHUNK
_SLOTS = 3
_NSUB = 16
_ROWS_PER = _N // _NSUB
_LANES = 16


def _triplet_columns(n):
    labels = list(range(int(n / 2))) + list(range(int(n / 2)))
    rng = np.random.RandomState(0)
    triplets = []
    for i in range(len(labels)):
        triplets_i = []
        for j in range(len(labels)):
            if labels[i] == labels[j] and i != j:
                for k in range(len(labels)):
                    if labels[i] != labels[k]:
                        triplets_i.append([i, j, k])
        rng.shuffle(triplets_i)
        triplets += triplets_i[:3]
    trip = np.asarray(triplets, dtype=np.int32)
    jj = np.zeros((_SLOTS, n), dtype=np.int32)
    kk = np.zeros((_SLOTS, n), dtype=np.int32)
    fill = np.zeros((n,), dtype=np.int64)
    for (i, j, k) in trip:
        m = fill[i]
        jj[m, i] = j
        kk[m, i] = k
        fill[i] += 1
    assert (fill == _SLOTS).all()
    return jj, kk, trip.shape[0]


_JJ, _KK, _NUM_TRIPLETS = _triplet_columns(_N)
_IDX_SC = np.zeros((_NSUB, 8, _ROWS_PER), dtype=np.int32)
for _m in range(_SLOTS):
    _IDX_SC[:, _m, :] = _JJ[_m].reshape(_NSUB, _ROWS_PER)
    _IDX_SC[:, 4 + _m, :] = _KK[_m].reshape(_NSUB, _ROWS_PER)
_IDX_SC_FLAT = _IDX_SC.reshape(-1)
_IDX_PER_SUB = 8 * _ROWS_PER


def _dist_kernel(x_ref, out_ref, acc_ref):
    d = pl.program_id(0)
    xc = x_ref[...].astype(jnp.bfloat16)
    part = jax.lax.dot_general(
        xc, xc,
        dimension_numbers=(((1,), (1,)), ((), ())),
        preferred_element_type=jnp.float32,
    )

    @pl.when(d == 0)
    def _init():
        acc_ref[...] = part

    @pl.when(d > 0)
    def _accum():
        acc_ref[...] += part

    @pl.when(d == _NCHUNKS - 1)
    def _epilogue():
        gram = acc_ref[...]
        rows = jax.lax.broadcasted_iota(jnp.int32, (_N, _N), 0)
        cols = jax.lax.broadcasted_iota(jnp.int32, (_N, _N), 1)
        diag = jnp.where(rows == cols, gram, 0.0)
        xn_col = jnp.sum(diag, axis=1, keepdims=True)
        xn_row = jnp.sum(diag, axis=0, keepdims=True)
        out_ref[...] = jnp.maximum(xn_col + xn_row - 2.0 * gram, 0.0)


_sc_mesh = plsc.VectorSubcoreMesh(
    core_axis_name="c", subcore_axis_name="s", num_cores=1, num_subcores=_NSUB
)


@functools.partial(
    pl.kernel,
    out_type=(
        jax.ShapeDtypeStruct((_NSUB, _LANES), jnp.float32),
        jax.ShapeDtypeStruct((_LANES,), jnp.float32),
    ),
    mesh=_sc_mesh,
    scratch_types=[
        pltpu.VMEM((_ROWS_PER, _N), jnp.float32),
        pltpu.VMEM((_IDX_PER_SUB,), jnp.int32),
        pltpu.VMEM((_LANES,), jnp.float32),
        pltpu.VMEM((_NSUB, _LANES), jnp.float32),
        pltpu.VMEM((_LANES,), jnp.float32),
    ],
    compiler_params=pltpu.CompilerParams(needs_layout_passes=False),
)
def _sc_loss(dist_hbm, idx_hbm, stage_hbm, out_hbm, rows_v, idx_v, acc_v, all_v, out_v):
    s = lax.axis_index("s")
    pltpu.sync_copy(dist_hbm.at[pl.ds(s * _ROWS_PER, _ROWS_PER), :], rows_v)
    pltpu.sync_copy(idx_hbm.at[pl.ds(s * _IDX_PER_SUB, _IDX_PER_SUB)], idx_v)

    acc = jnp.zeros((_LANES,), jnp.float32)
    for g in range(_ROWS_PER // _LANES):
        row_idx = lax.iota(jnp.int32, _LANES) + g * _LANES
        for m in range(_SLOTS):
            jj = idx_v[pl.ds(m * _ROWS_PER + g * _LANES, _LANES)]
            kk = idx_v[pl.ds((4 + m) * _ROWS_PER + g * _LANES, _LANES)]
            dj = plsc.load_gather(rows_v, [row_idx, jj])
            dk = plsc.load_gather(rows_v, [row_idx, kk])
            delta = dj - dk
            t = jnp.exp(-jnp.abs(delta))
            z = t / (2.0 + t)
            z2 = z * z
            log1p = 2.0 * z * (1.0 + z2 * (1.0 / 3.0 + z2 * 0.2))
            acc = acc + jnp.maximum(delta, 0.0) + log1p

    acc_v[...] = acc
    pltpu.sync_copy(acc_v, stage_hbm.at[s])
    plsc.subcore_barrier()

    @pl.when(s == 0)
    def _reduce():
        pltpu.sync_copy(stage_hbm, all_v)
        tot = jnp.zeros((_LANES,), jnp.float32)
        for r in range(_NSUB):
            tot = tot + all_v[r, :]
        loss = jnp.sum(tot, axis=0) * (1.0 / float(_NUM_TRIPLETS))
        out_v[...] = jnp.full((_LANES,), loss, dtype=jnp.float32)
        pltpu.sync_copy(out_v, out_hbm)


@jax.jit
def kernel(x):
    dist = pl.pallas_call(
        _dist_kernel,
        grid=(_NCHUNKS,),
        out_shape=jax.ShapeDtypeStruct((_N, _N), jnp.float32),
        in_specs=[pl.BlockSpec((_N, _CHUNK), lambda d: (0, d))],
        out_specs=pl.BlockSpec((_N, _N), lambda d: (0, 0)),
        scratch_shapes=[pltpu.VMEM((_N, _N), jnp.float32)],
    )(x)
    _, out16 = _sc_loss(dist, jnp.asarray(_IDX_SC_FLAT))
    return out16[0:1]
